# baseline (speedup 1.0000x reference)
import jax
import jax.numpy as jnp
from jax import lax
from jax.experimental import pallas as pl
from jax.experimental.pallas import tpu as pltpu
from jax.experimental.pallas import tpu_sc as plsc

_P, _N, _E, _D = 2, 10000, 160000, 128
_L = 3
_NEXP, _NATT = 8, 64
_NS = 16
_NC = 2
_CH = 96
_NCHUNK = 108
_EPT = _CH * _NCHUNK
_EPAD = _EPT * _NS
_NDUMP = 16
_NACC = _N + _NDUMP
_G = 4
_IB = 6
_LCM = 12
_RPT = _N // _NS


def _seg_body(x_hbm, idx_hbm, out_hbm,
              idxv, rows, i0, i1, i2, i3, i4, i5, g0, g1, g2, g3, acc):
    isems = (i0, i1, i2, i3, i4, i5)
    gsems = (g0, g1, g2, g3)
    c = lax.axis_index("c")
    s = lax.axis_index("s")
    wid = c * _NS + s
    r0 = s * _RPT
    pltpu.sync_copy(x_hbm.at[pl.ds(c * _N + r0, _RPT)], acc.at[pl.ds(r0, _RPT)])

    for k in range(_IB):
        pltpu.async_copy(idx_hbm.at[wid, k], idxv.at[k], isems[k])
    plsc.subcore_barrier()
    for g in range(_G):
        pltpu.make_async_copy(idx_hbm.at[wid, g], idxv.at[g], isems[g]).wait()
        pltpu.async_copy(x_hbm.at[idxv.at[g, 0]], rows.at[g], gsems[g])

    def step(j, carry):
        base = j * _LCM
        for b in range(_LCM):
            i = base + b
            ib = b % _IB
            rb = b % _G
            pltpu.make_async_copy(x_hbm.at[idxv.at[ib, 0]], rows.at[rb],
                                  gsems[rb]).wait()
            pltpu.sync_copy(rows.at[rb], acc.at[idxv.at[ib, 1]], add=True)

            @pl.when(i + _IB < _NCHUNK)
            def _():
                pltpu.async_copy(idx_hbm.at[wid, i + _IB], idxv.at[ib],
                                 isems[ib])

            @pl.when(i + _G < _NCHUNK)
            def _():
                bb = (b + _G) % _IB
                pltpu.make_async_copy(idx_hbm.at[wid, i + _G], idxv.at[bb],
                                      isems[bb]).wait()
                pltpu.async_copy(x_hbm.at[idxv.at[bb, 0]], rows.at[rb],
                                 gsems[rb])
        return carry

    lax.fori_loop(0, _NCHUNK // _LCM, step, 0)

    plsc.subcore_barrier()
    pltpu.sync_copy(acc.at[pl.ds(r0, _RPT)], out_hbm.at[pl.ds(c * _N + r0, _RPT)])


_seg_call = None


def _seg(x2, idx):
    global _seg_call
    if _seg_call is None:
        _seg_call = pl.kernel(
            _seg_body,
            out_type=jax.ShapeDtypeStruct((_P * _N, _D), jnp.float32),
            mesh=plsc.VectorSubcoreMesh(core_axis_name="c",
                                        subcore_axis_name="s",
                                        num_cores=_NC, num_subcores=_NS),
            compiler_params=pltpu.CompilerParams(use_tc_tiling_on_sc=False),
            scratch_types=(
                [pltpu.VMEM((_IB, 2, _CH), jnp.int32),
                 pltpu.VMEM((_G, _CH, _D), jnp.float32)]
                + [pltpu.SemaphoreType.DMA] * (_IB + _G)
                + [pltpu.VMEM_SHARED((_NACC, _D), jnp.float32)]
            ),
        )
    return _seg_call(x2, idx)


_ROWB = 2000
_NBLK = (_P * _N) // _ROWB
_BPG = _N // _ROWB


def _ln(t, g, b):
    mu = jnp.mean(t, axis=-1, keepdims=True)
    var = jnp.mean((t - mu) * (t - mu), axis=-1, keepdims=True)
    return (t - mu) * lax.rsqrt(var + 1e-5) * g + b


def _mlp_mid_body(hp_ref, idn_ref, wa_ref, ba_ref, wb_ref, bb_ref,
                  g_ref, b_ref, o_ref):
    h1 = jnp.maximum(
        jnp.dot(hp_ref[...], wa_ref[...], preferred_element_type=jnp.float32)
        + ba_ref[...], 0.0)
    h2 = (jnp.dot(h1, wb_ref[...], preferred_element_type=jnp.float32)
          + bb_ref[...] + idn_ref[...])
    o_ref[...] = _ln(h2, g_ref[...], b_ref[...])


def _mlp_last_body(hp_ref, wa_ref, ba_ref, wb_ref, bb_ref,
                   g_ref, b_ref, gsum_ref):
    i = pl.program_id(0)
    h1 = jnp.maximum(
        jnp.dot(hp_ref[...], wa_ref[...], preferred_element_type=jnp.float32)
        + ba_ref[...], 0.0)
    h2 = (jnp.dot(h1, wb_ref[...], preferred_element_type=jnp.float32)
          + bb_ref[...])
    x = _ln(h2, g_ref[...], b_ref[...])

    @pl.when(i % _BPG == 0)
    def _():
        gsum_ref[...] = jnp.zeros_like(gsum_ref)

    gsum_ref[...] += jnp.sum(x, axis=0, keepdims=True)[None]


_w_spec = pl.BlockSpec((_D, _D), lambda i: (0, 0))
_v_spec = pl.BlockSpec((1, _D), lambda i: (0, 0))
_row_spec = pl.BlockSpec((_ROWB, _D), lambda i: (i, 0))

_mlp_mid = pl.pallas_call(
    _mlp_mid_body,
    grid=(_NBLK,),
    in_specs=[_row_spec, _row_spec, _w_spec, _v_spec, _w_spec, _v_spec,
              _v_spec, _v_spec],
    out_specs=_row_spec,
    out_shape=jax.ShapeDtypeStruct((_P * _N, _D), jnp.float32),
)

_mlp_last = pl.pallas_call(
    _mlp_last_body,
    grid=(_NBLK,),
    in_specs=[_row_spec, _w_spec, _v_spec, _w_spec, _v_spec, _v_spec,
              _v_spec],
    out_specs=pl.BlockSpec((1, 1, _D), lambda i: (i // _BPG, 0, 0)),
    out_shape=jax.ShapeDtypeStruct((_P, 1, _D), jnp.float32),
)


def _pool_body(g_ref, w1t_ref, w2t_ref, o_ref):
    g = g_ref[...]
    sup = jnp.tanh(jnp.dot(g, w1t_ref[...],
                           preferred_element_type=jnp.float32))
    logits = jnp.dot(sup, w2t_ref[...],
                     preferred_element_type=jnp.float32)
    row = lax.broadcasted_iota(jnp.int32, (8, _NEXP), 0)
    logits = jnp.where(row < _P, logits, -1e30)
    m = jnp.max(logits, axis=0, keepdims=True)
    e = jnp.exp(logits - m)
    att = e / jnp.sum(e, axis=0, keepdims=True)
    o_ref[...] = lax.dot_general(att, g, (((0,), (0,)), ((), ())),
                                 preferred_element_type=jnp.float32)


_pool = pl.pallas_call(
    _pool_body,
    in_specs=[pl.BlockSpec((8, _D), lambda: (0, 0)),
              pl.BlockSpec((_D, _NATT), lambda: (0, 0)),
              pl.BlockSpec((_NATT, _NEXP), lambda: (0, 0))],
    out_specs=pl.BlockSpec((_NEXP, _D), lambda: (0, 0)),
    out_shape=jax.ShapeDtypeStruct((_NEXP, _D), jnp.float32),
)


def kernel(decomp_x, decomp_edge_index, decomp_edge_attr,
           Wa, ba, Wb, bb, lng, lnb, attw1, attw2):
    del decomp_edge_attr
    x2 = decomp_x.reshape(_P * _N, _D)
    offs = (jnp.arange(_P, dtype=jnp.int32) * _N)[:, None]
    padi = jnp.arange(_EPAD - _E, dtype=jnp.int32)
    si = jnp.concatenate(
        [decomp_edge_index[:, 0, :],
         jnp.broadcast_to(padi % _N, (_P, padi.size))], axis=1)
    di = jnp.concatenate(
        [decomp_edge_index[:, 1, :],
         jnp.broadcast_to(_N + (padi % _NDUMP), (_P, padi.size))], axis=1)
    src = (si + offs).reshape(_NC * _NS, _NCHUNK, _CH)
    dst = di.reshape(_NC * _NS, _NCHUNK, _CH)
    idx = jnp.stack([src, dst], axis=2)

    g = None
    for l in range(_L):
        hp = _seg(x2, idx)
        ba_l, bb_l = ba[l].reshape(1, _D), bb[l].reshape(1, _D)
        g_l, b_l = lng[l].reshape(1, _D), lnb[l].reshape(1, _D)
        if l < _L - 1:
            x2 = _mlp_mid(hp, x2, Wa[l], ba_l, Wb[l], bb_l, g_l, b_l)
        else:
            g = _mlp_last(hp, Wa[l], ba_l, Wb[l], bb_l, g_l, b_l)

    g8 = jnp.zeros((8, _D), jnp.float32).at[:_P].set(g.reshape(_P, _D))
    out8 = _pool(g8, attw1.T, attw2.T)
    return out8.reshape(1, _NEXP * _D)


# speedup vs baseline: 1.0937x; 1.0937x over previous
import jax
import jax.numpy as jnp
from jax import lax
from jax.experimental import pallas as pl
from jax.experimental.pallas import tpu as pltpu
from jax.experimental.pallas import tpu_sc as plsc

_P, _N, _E, _D = 2, 10000, 160000, 128
_L = 3
_NEXP, _NATT = 8, 64
_NS = 16
_NC = 2
_CH = 80
_EPT = _E // _NS
_NCHUNK = _EPT // _CH
_G = 4
_IB = 6
_LCM = 12
_NLOOP = _NCHUNK // _LCM
_TAIL = _NCHUNK - _NLOOP * _LCM
_RPT = _N // _NS


def _seg_body(x_hbm, ei_hbm, out_hbm,
              sidx, didx, rows, i0, i1, i2, i3, i4, i5, g0, g1, g2, g3, acc):
    isems = (i0, i1, i2, i3, i4, i5)
    gsems = (g0, g1, g2, g3)
    c = lax.axis_index("c")
    s = lax.axis_index("s")
    ebase = s * _EPT
    xoff = c * _N
    r0 = s * _RPT

    def idx_issue(k, slot):
        pltpu.async_copy(ei_hbm.at[c, 0, pl.ds(ebase + k * _CH, _CH)],
                         sidx.at[slot], isems[slot])
        pltpu.async_copy(ei_hbm.at[c, 1, pl.ds(ebase + k * _CH, _CH)],
                         didx.at[slot], isems[slot])

    def gather_issue(slot, rslot):
        pltpu.make_async_copy(ei_hbm.at[c, 0, pl.ds(0, _CH)],
                              sidx.at[slot], isems[slot]).wait()
        pltpu.make_async_copy(ei_hbm.at[c, 0, pl.ds(0, _CH)],
                              didx.at[slot], isems[slot]).wait()
        for q in range(_CH // 16):
            sl = pl.ds(q * 16, 16)
            sidx[slot, sl] = sidx[slot, sl] + xoff
        pltpu.async_copy(x_hbm.at[sidx.at[slot]], rows.at[rslot],
                         gsems[rslot])

    def turn(i, b):
        ib = b % _IB
        rb = b % _G
        pltpu.make_async_copy(x_hbm.at[sidx.at[ib]], rows.at[rb],
                              gsems[rb]).wait()
        pltpu.sync_copy(rows.at[rb], acc.at[didx.at[ib]], add=True)

        @pl.when(i + _IB < _NCHUNK)
        def _():
            idx_issue(i + _IB, ib)

        @pl.when(i + _G < _NCHUNK)
        def _():
            gather_issue((b + _G) % _IB, rb)

    for k in range(_IB):
        idx_issue(k, k)
    pltpu.sync_copy(x_hbm.at[pl.ds(xoff + r0, _RPT)], acc.at[pl.ds(r0, _RPT)])
    plsc.subcore_barrier()
    for g in range(_G):
        gather_issue(g, g)

    def step(j, carry):
        base = j * _LCM
        for b in range(_LCM):
            turn(base + b, b)
        return carry

    lax.fori_loop(0, _NLOOP, step, 0)
    for k in range(_TAIL):
        turn(_NLOOP * _LCM + k, k)

    plsc.subcore_barrier()
    pltpu.sync_copy(acc.at[pl.ds(r0, _RPT)], out_hbm.at[pl.ds(c * _N + r0, _RPT)])


_seg_call = None


def _seg(x2, ei):
    global _seg_call
    if _seg_call is None:
        _seg_call = pl.kernel(
            _seg_body,
            out_type=jax.ShapeDtypeStruct((_P * _N, _D), jnp.float32),
            mesh=plsc.VectorSubcoreMesh(core_axis_name="c",
                                        subcore_axis_name="s",
                                        num_cores=_NC, num_subcores=_NS),
            compiler_params=pltpu.CompilerParams(use_tc_tiling_on_sc=False),
            scratch_types=(
                [pltpu.VMEM((_IB, _CH), jnp.int32),
                 pltpu.VMEM((_IB, _CH), jnp.int32),
                 pltpu.VMEM((_G, _CH, _D), jnp.float32)]
                + [pltpu.SemaphoreType.DMA] * (_IB + _G)
                + [pltpu.VMEM_SHARED((_N, _D), jnp.float32)]
            ),
        )
    return _seg_call(x2, ei)


_ROWB = 2000
_NBLK = (_P * _N) // _ROWB
_BPG = _N // _ROWB


def _ln(t, g, b):
    mu = jnp.mean(t, axis=-1, keepdims=True)
    var = jnp.mean((t - mu) * (t - mu), axis=-1, keepdims=True)
    return (t - mu) * lax.rsqrt(var + 1e-5) * g + b


def _mlp_mid_body(hp_ref, idn_ref, wa_ref, ba_ref, wb_ref, bb_ref,
                  g_ref, b_ref, o_ref):
    h1 = jnp.maximum(
        jnp.dot(hp_ref[...], wa_ref[...], preferred_element_type=jnp.float32)
        + ba_ref[...], 0.0)
    h2 = (jnp.dot(h1, wb_ref[...], preferred_element_type=jnp.float32)
          + bb_ref[...] + idn_ref[...])
    o_ref[...] = _ln(h2, g_ref[...], b_ref[...])


def _mlp_last_body(hp_ref, wa_ref, ba_ref, wb_ref, bb_ref,
                   g_ref, b_ref, gsum_ref):
    i = pl.program_id(0)
    h1 = jnp.maximum(
        jnp.dot(hp_ref[...], wa_ref[...], preferred_element_type=jnp.float32)
        + ba_ref[...], 0.0)
    h2 = (jnp.dot(h1, wb_ref[...], preferred_element_type=jnp.float32)
          + bb_ref[...])
    x = _ln(h2, g_ref[...], b_ref[...])

    @pl.when(i % _BPG == 0)
    def _():
        gsum_ref[...] = jnp.zeros_like(gsum_ref)

    gsum_ref[...] += jnp.sum(x, axis=0, keepdims=True)[None]


_w_spec = pl.BlockSpec((_D, _D), lambda i: (0, 0))
_v_spec = pl.BlockSpec((1, _D), lambda i: (0, 0))
_row_spec = pl.BlockSpec((_ROWB, _D), lambda i: (i, 0))

_mlp_mid = pl.pallas_call(
    _mlp_mid_body,
    grid=(_NBLK,),
    in_specs=[_row_spec, _row_spec, _w_spec, _v_spec, _w_spec, _v_spec,
              _v_spec, _v_spec],
    out_specs=_row_spec,
    out_shape=jax.ShapeDtypeStruct((_P * _N, _D), jnp.float32),
)

_mlp_last = pl.pallas_call(
    _mlp_last_body,
    grid=(_NBLK,),
    in_specs=[_row_spec, _w_spec, _v_spec, _w_spec, _v_spec, _v_spec,
              _v_spec],
    out_specs=pl.BlockSpec((1, 1, _D), lambda i: (i // _BPG, 0, 0)),
    out_shape=jax.ShapeDtypeStruct((_P, 1, _D), jnp.float32),
)


def _pool_body(g_ref, w1t_ref, w2t_ref, o_ref):
    g = g_ref[...]
    sup = jnp.tanh(jnp.dot(g, w1t_ref[...],
                           preferred_element_type=jnp.float32))
    logits = jnp.dot(sup, w2t_ref[...],
                     preferred_element_type=jnp.float32)
    row = lax.broadcasted_iota(jnp.int32, (8, _NEXP), 0)
    logits = jnp.where(row < _P, logits, -1e30)
    m = jnp.max(logits, axis=0, keepdims=True)
    e = jnp.exp(logits - m)
    att = e / jnp.sum(e, axis=0, keepdims=True)
    o_ref[...] = lax.dot_general(att, g, (((0,), (0,)), ((), ())),
                                 preferred_element_type=jnp.float32)


_pool = pl.pallas_call(
    _pool_body,
    in_specs=[pl.BlockSpec((8, _D), lambda: (0, 0)),
              pl.BlockSpec((_D, _NATT), lambda: (0, 0)),
              pl.BlockSpec((_NATT, _NEXP), lambda: (0, 0))],
    out_specs=pl.BlockSpec((_NEXP, _D), lambda: (0, 0)),
    out_shape=jax.ShapeDtypeStruct((_NEXP, _D), jnp.float32),
)


def kernel(decomp_x, decomp_edge_index, decomp_edge_attr,
           Wa, ba, Wb, bb, lng, lnb, attw1, attw2):
    del decomp_edge_attr
    x2 = decomp_x.reshape(_P * _N, _D)

    g = None
    for l in range(_L):
        hp = _seg(x2, decomp_edge_index)
        ba_l, bb_l = ba[l].reshape(1, _D), bb[l].reshape(1, _D)
        g_l, b_l = lng[l].reshape(1, _D), lnb[l].reshape(1, _D)
        if l < _L - 1:
            x2 = _mlp_mid(hp, x2, Wa[l], ba_l, Wb[l], bb_l, g_l, b_l)
        else:
            g = _mlp_last(hp, Wa[l], ba_l, Wb[l], bb_l, g_l, b_l)

    g8 = jnp.zeros((8, _D), jnp.float32).at[:_P].set(g.reshape(_P, _D))
    out8 = _pool(g8, attw1.T, attw2.T)
    return out8.reshape(1, _NEXP * _D)
